# Initial kernel scaffold; baseline (speedup 1.0000x reference)
import jax
import jax.numpy as jnp
from jax.experimental import pallas as pl


def kernel(x, adj, W1, b1, W2, b2):
    raise NotImplementedError("write your pallas kernel here")


# speedup vs baseline: 1.0110x; 1.0110x over previous
import functools

import jax
import jax.numpy as jnp
from jax.experimental import pallas as pl
from jax.experimental.pallas import tpu as pltpu


def _body(x_ref, adj_ref, w1_ref, b1_ref, w2_ref, b2_ref, out_ref,
          s1_ref, s2_ref, *, bm, phase_steps):
    i = pl.program_id(0)

    @pl.when(i == 0)
    def _():
        s1_ref[...] = jnp.dot(x_ref[...], w1_ref[...],
                              preferred_element_type=jnp.float32)

    @pl.when(i < phase_steps)
    def _():
        h = jnp.dot(adj_ref[...], s1_ref[...],
                    preferred_element_type=jnp.float32) + b1_ref[...]
        h = jnp.maximum(h, 0.0)
        row = jnp.dot(h, w2_ref[...], preferred_element_type=jnp.float32)
        s2_ref[pl.ds(i * bm, bm), :] = row

    @pl.when(i >= phase_steps)
    def _():
        o = jnp.dot(adj_ref[...], s2_ref[...],
                    preferred_element_type=jnp.float32) + b2_ref[...]
        shifted = o - jnp.max(o, axis=-1, keepdims=True)
        lse = jnp.log(jnp.sum(jnp.exp(shifted), axis=-1, keepdims=True))
        out_ref[...] = shifted - lse


def kernel(x, adj, W1, b1, W2, b2):
    n, nfeat = x.shape
    nhid = W1.shape[1]
    nclass = W2.shape[1]

    bm = next(b for b in (400, 200, 80, 40, 8) if n % b == 0)
    phase_steps = n // bm
    grid = (2 * phase_steps,)

    b1_2d = b1.reshape(1, nhid)
    b2_2d = b2.reshape(1, nclass)

    out = pl.pallas_call(
        functools.partial(_body, bm=bm, phase_steps=phase_steps),
        grid=grid,
        in_specs=[
            pl.BlockSpec((n, nfeat), lambda i: (0, 0)),
            pl.BlockSpec((bm, n), lambda i, ps=phase_steps: (jax.lax.rem(i, ps), 0)),
            pl.BlockSpec((nfeat, nhid), lambda i: (0, 0)),
            pl.BlockSpec((1, nhid), lambda i: (0, 0)),
            pl.BlockSpec((nhid, nclass), lambda i: (0, 0)),
            pl.BlockSpec((1, nclass), lambda i: (0, 0)),
        ],
        out_specs=pl.BlockSpec(
            (bm, nclass),
            lambda i, ps=phase_steps: (jax.lax.max(i - ps, 0), 0)),
        out_shape=jax.ShapeDtypeStruct((n, nclass), jnp.float32),
        scratch_shapes=[
            pltpu.VMEM((n, nhid), jnp.float32),
            pltpu.VMEM((n, nclass), jnp.float32),
        ],
        compiler_params=pltpu.CompilerParams(
            dimension_semantics=("arbitrary",),
        ),
    )(x, adj, W1, b1_2d, W2, b2_2d)
    return out
